# Initial kernel scaffold; baseline (speedup 1.0000x reference)
import jax
import jax.numpy as jnp
from jax.experimental import pallas as pl


def kernel(x, edge_index, index, W1, b1, W2, b2, W3, b3):
    raise NotImplementedError("write your pallas kernel here")


# speedup vs baseline: 6.3998x; 6.3998x over previous
import functools

import jax
import jax.numpy as jnp
from jax import lax
from jax.experimental import pallas as pl
from jax.experimental.pallas import tpu as pltpu
from jax.experimental.pallas import tpu_sc as plsc

N = 10000
E = 160000
NP = 10240
EP = 163840
EROWS = 1280
IDX = 2048
PAD_NODE = N

NC = 2
NS = 16
ROWS_T = NP // NS
ER_SC = EROWS // NS
ER_ALL = EROWS // (NC * NS)

_MESH = plsc.VectorSubcoreMesh(core_axis_name="c", subcore_axis_name="s")


def _f32(shape):
    return jax.ShapeDtypeStruct(shape, jnp.float32)


@functools.partial(
    pl.kernel,
    out_type=(_f32((NP,)), _f32((NP,))),
    mesh=_MESH,
    scratch_types=[
        pltpu.VMEM((ER_ALL, 128), jnp.int32),
        pltpu.VMEM((128,), jnp.float32),
        pltpu.VMEM_SHARED((NP,), jnp.float32),
    ],
)
def _sc_degree(dst_hbm, zeros_hbm, deg_a, deg_b, didx_v, ones_v, sdeg):
    c = lax.axis_index("c")
    s = lax.axis_index("s")
    wid = c * NS + s
    for i in range(8):
        ones_v[pl.ds(i * 16, 16)] = jnp.ones((16,), jnp.float32)

    @pl.when(s == 0)
    def _():
        pltpu.sync_copy(zeros_hbm, sdeg)

    pltpu.sync_copy(dst_hbm.at[pl.ds(wid * ER_ALL, ER_ALL)], didx_v)
    plsc.subcore_barrier()

    def body(j, carry):
        pltpu.sync_copy(ones_v, sdeg.at[didx_v.at[j]], add=True)
        return carry

    lax.fori_loop(0, ER_ALL, body, 0)
    plsc.subcore_barrier()
    for ci, out_ref in enumerate((deg_a, deg_b)):
        @pl.when(c == ci)
        def _(out_ref=out_ref):
            pltpu.sync_copy(sdeg.at[pl.ds(s * ROWS_T, ROWS_T)],
                            out_ref.at[pl.ds(s * ROWS_T, ROWS_T)])


@functools.partial(
    pl.kernel,
    out_type=(_f32((NP, 128)), _f32((NP, 128))),
    mesh=_MESH,
    scratch_types=[
        pltpu.VMEM((ER_SC, 128), jnp.int32),
        pltpu.VMEM((ER_SC, 128), jnp.int32),
        pltpu.VMEM((128, 128), jnp.float32),
        pltpu.VMEM_SHARED((NP, 128), jnp.float32),
        pltpu.SemaphoreType.DMA,
    ],
)
def _sc_scatter_split(g_lo, g_hi, src_hbm, dst_hbm, s_lo, s_hi,
                      sidx_v, didx_v, rows_v, acc, sem):
    c = lax.axis_index("c")
    s = lax.axis_index("s")
    pltpu.sync_copy(src_hbm.at[pl.ds(s * ER_SC, ER_SC)], sidx_v)
    pltpu.sync_copy(dst_hbm.at[pl.ds(s * ER_SC, ER_SC)], didx_v)
    for ci, (g_ref, s_ref) in enumerate(((g_lo, s_lo), (g_hi, s_hi))):
        @pl.when(c == ci)
        def _(g_ref=g_ref, s_ref=s_ref):
            pltpu.sync_copy(g_ref.at[pl.ds(s * ROWS_T, ROWS_T)],
                            acc.at[pl.ds(s * ROWS_T, ROWS_T)])
            plsc.subcore_barrier()

            def body(j, carry):
                pltpu.async_copy(g_ref.at[sidx_v.at[j]], rows_v, sem).wait()
                pltpu.sync_copy(rows_v, acc.at[didx_v.at[j]], add=True)
                return carry

            lax.fori_loop(0, ER_SC, body, 0)
            plsc.subcore_barrier()
            pltpu.sync_copy(acc.at[pl.ds(s * ROWS_T, ROWS_T)],
                            s_ref.at[pl.ds(s * ROWS_T, ROWS_T)])


@functools.partial(
    pl.kernel,
    out_type=(_f32((NP, 128)), _f32((NP, 128))),
    mesh=_MESH,
    scratch_types=[
        pltpu.VMEM((ER_ALL, 128), jnp.int32),
        pltpu.VMEM((ER_ALL, 128), jnp.int32),
        pltpu.VMEM((128, 128), jnp.float32),
        pltpu.VMEM_SHARED((NP, 128), jnp.float32),
        pltpu.SemaphoreType.DMA,
    ],
)
def _sc_scatter_edges(g_hbm, zeros_hbm, src_hbm, dst_hbm, s_a, s_b,
                      sidx_v, didx_v, rows_v, acc, sem):
    c = lax.axis_index("c")
    s = lax.axis_index("s")
    wid = c * NS + s
    pltpu.sync_copy(src_hbm.at[pl.ds(wid * ER_ALL, ER_ALL)], sidx_v)
    pltpu.sync_copy(dst_hbm.at[pl.ds(wid * ER_ALL, ER_ALL)], didx_v)
    for ci, (init_ref, s_ref) in enumerate(((g_hbm, s_a), (zeros_hbm, s_b))):
        @pl.when(c == ci)
        def _(init_ref=init_ref, s_ref=s_ref):
            pltpu.sync_copy(init_ref.at[pl.ds(s * ROWS_T, ROWS_T)],
                            acc.at[pl.ds(s * ROWS_T, ROWS_T)])
            plsc.subcore_barrier()

            def body(j, carry):
                pltpu.async_copy(g_hbm.at[sidx_v.at[j]], rows_v, sem).wait()
                pltpu.sync_copy(rows_v, acc.at[didx_v.at[j]], add=True)
                return carry

            lax.fori_loop(0, ER_ALL, body, 0)
            plsc.subcore_barrier()
            pltpu.sync_copy(acc.at[pl.ds(s * ROWS_T, ROWS_T)],
                            s_ref.at[pl.ds(s * ROWS_T, ROWS_T)])


_IPT = IDX // 32


@functools.partial(
    pl.kernel,
    out_type=_f32((IDX, 64)),
    mesh=_MESH,
    scratch_types=[
        pltpu.VMEM((_IPT,), jnp.int32),
        pltpu.VMEM((_IPT, 128), jnp.float32),
        pltpu.SemaphoreType.DMA,
    ],
)
def _sc_gather_out(h3_hbm, idx_hbm, out_hbm, iidx_v, orows_v, sem):
    c = lax.axis_index("c")
    s = lax.axis_index("s")
    base = (c * NS + s) * _IPT
    pltpu.sync_copy(idx_hbm.at[pl.ds(base, _IPT)], iidx_v)
    pltpu.async_copy(h3_hbm.at[iidx_v], orows_v, sem).wait()

    def body(r, carry):
        pltpu.sync_copy(orows_v.at[r, pl.ds(0, 64)], out_hbm.at[base + r])
        return carry

    lax.fori_loop(0, _IPT, body, 0)


_BM = 1024
_PREC = jax.lax.Precision.HIGHEST


def _tc1_body(x_ref, w_ref, da_ref, db_ref, glo_ref, ghi_ref, dv_ref):
    dv = jax.lax.rsqrt(da_ref[...] + db_ref[...] + 1.0)[:, None]
    m = jnp.dot(x_ref[...], w_ref[...], preferred_element_type=jnp.float32,
                precision=_PREC)
    g = m * dv
    glo_ref[...] = g[:, :128]
    ghi_ref[...] = g[:, 128:]
    dv_ref[...] = dv


def _tc1(x_p, w1, deg_a, deg_b):
    d_in = x_p.shape[1]
    return pl.pallas_call(
        _tc1_body,
        grid=(NP // _BM,),
        in_specs=[
            pl.BlockSpec((_BM, d_in), lambda m: (m, 0)),
            pl.BlockSpec((d_in, 256), lambda m: (0, 0)),
            pl.BlockSpec((_BM,), lambda m: (m,)),
            pl.BlockSpec((_BM,), lambda m: (m,)),
        ],
        out_specs=(
            pl.BlockSpec((_BM, 128), lambda m: (m, 0)),
            pl.BlockSpec((_BM, 128), lambda m: (m, 0)),
            pl.BlockSpec((_BM, 1), lambda m: (m, 0)),
        ),
        out_shape=(_f32((NP, 128)), _f32((NP, 128)), _f32((NP, 1))),
    )(x_p, w1, deg_a, deg_b)


def _tc_mid_body(combine, pad_out, sa_ref, sb_ref, dv_ref, b_ref, w_ref,
                 g_ref):
    dv = dv_ref[...]
    if combine == "concat":
        sfull = jnp.concatenate([sa_ref[...], sb_ref[...]], axis=1)
    else:
        sfull = sa_ref[...] + sb_ref[...]
    h = jnp.maximum(dv * sfull + b_ref[...][None, :], 0.0)
    m = jnp.dot(h, w_ref[...], preferred_element_type=jnp.float32,
                precision=_PREC)
    g = m * dv
    if pad_out:
        g = jnp.concatenate([g, jnp.zeros_like(g)], axis=1)
    g_ref[...] = g


def _tc_mid(s_a, s_b, dinv, b, w, combine):
    dh_in = s_a.shape[1]
    d_in = b.shape[0]
    d_out = w.shape[1]
    pad_out = d_out < 128
    d_store = 128
    return pl.pallas_call(
        functools.partial(_tc_mid_body, combine, pad_out),
        grid=(NP // _BM,),
        in_specs=[
            pl.BlockSpec((_BM, dh_in), lambda m: (m, 0)),
            pl.BlockSpec((_BM, dh_in), lambda m: (m, 0)),
            pl.BlockSpec((_BM, 1), lambda m: (m, 0)),
            pl.BlockSpec((d_in,), lambda m: (0,)),
            pl.BlockSpec((d_in, d_out), lambda m: (0, 0)),
        ],
        out_specs=pl.BlockSpec((_BM, d_store), lambda m: (m, 0)),
        out_shape=_f32((NP, d_store)),
    )(s_a, s_b, dinv, b, w)


def _tc_final_body(sa_ref, sb_ref, dv_ref, b_ref, h_ref):
    sfull = sa_ref[...] + sb_ref[...]
    h_ref[...] = jnp.maximum(dv_ref[...] * sfull + b_ref[...][None, :], 0.0)


def _tc_final(s_a, s_b, dinv, b_pad):
    return pl.pallas_call(
        _tc_final_body,
        grid=(NP // _BM,),
        in_specs=[
            pl.BlockSpec((_BM, 128), lambda m: (m, 0)),
            pl.BlockSpec((_BM, 128), lambda m: (m, 0)),
            pl.BlockSpec((_BM, 1), lambda m: (m, 0)),
            pl.BlockSpec((128,), lambda m: (0,)),
        ],
        out_specs=pl.BlockSpec((_BM, 128), lambda m: (m, 0)),
        out_shape=_f32((NP, 128)),
    )(s_a, s_b, dinv, b_pad)


def kernel(x, edge_index, index, W1, b1, W2, b2, W3, b3):
    pad = jnp.full((EP - E,), PAD_NODE, dtype=jnp.int32)
    src2 = jnp.concatenate([edge_index[0], pad]).reshape(EROWS, 128)
    dst2 = jnp.concatenate([edge_index[1], pad]).reshape(EROWS, 128)
    x_p = jnp.pad(x, ((0, NP - N), (0, 0)))
    zeros_np = jnp.zeros((NP,), jnp.float32)
    zeros_128 = jnp.zeros((NP, 128), jnp.float32)
    b3_pad = jnp.pad(b3, (0, 64))

    deg_a, deg_b = _sc_degree(dst2, zeros_np)
    g1_lo, g1_hi, dinv = _tc1(x_p, W1, deg_a, deg_b)
    s1_lo, s1_hi = _sc_scatter_split(g1_lo, g1_hi, src2, dst2)
    g2 = _tc_mid(s1_lo, s1_hi, dinv, b1, W2, combine="concat")
    s2_a, s2_b = _sc_scatter_edges(g2, zeros_128, src2, dst2)
    g3 = _tc_mid(s2_a, s2_b, dinv, b2, W3, combine="add")
    s3_a, s3_b = _sc_scatter_edges(g3, zeros_128, src2, dst2)
    h3 = _tc_final(s3_a, s3_b, dinv, b3_pad)
    return _sc_gather_out(h3, index)


# speedup vs baseline: 8.4225x; 1.3161x over previous
import functools

import jax
import jax.numpy as jnp
from jax import lax
from jax.experimental import pallas as pl
from jax.experimental.pallas import tpu as pltpu
from jax.experimental.pallas import tpu_sc as plsc

N = 10000
E = 160000
NP = 10240
EP = 163840
EC = 64
EROWS = EP // EC
_K = 5
NSUP = EROWS // _K
IDX = 2048
PAD_NODE = N

NC = 2
NS = 16
ROWS_T = NP // NS
SUP_SC = NSUP // NS
SUP_ALL = NSUP // (NC * NS)

_MESH = plsc.VectorSubcoreMesh(core_axis_name="c", subcore_axis_name="s")


def _f32(shape):
    return jax.ShapeDtypeStruct(shape, jnp.float32)


@functools.partial(
    pl.kernel,
    out_type=(_f32((NP,)), _f32((NP,))),
    mesh=_MESH,
    scratch_types=[
        pltpu.VMEM((SUP_ALL, _K, EC), jnp.int32),
        pltpu.VMEM((EC,), jnp.float32),
        pltpu.VMEM_SHARED((NP,), jnp.float32),
    ],
)
def _sc_degree(dst_hbm, zeros_hbm, deg_a, deg_b, didx_v, ones_v, sdeg):
    c = lax.axis_index("c")
    s = lax.axis_index("s")
    wid = c * NS + s
    for i in range(EC // 16):
        ones_v[pl.ds(i * 16, 16)] = jnp.ones((16,), jnp.float32)

    @pl.when(s == 0)
    def _():
        pltpu.sync_copy(zeros_hbm, sdeg)

    pltpu.sync_copy(dst_hbm.at[pl.ds(wid * SUP_ALL, SUP_ALL)], didx_v)
    plsc.subcore_barrier()

    def body(j, carry):
        for t in range(_K):
            pltpu.sync_copy(ones_v, sdeg.at[didx_v.at[j, t]], add=True)
        return carry

    lax.fori_loop(0, SUP_ALL, body, 0)
    plsc.subcore_barrier()
    for ci, out_ref in enumerate((deg_a, deg_b)):
        @pl.when(c == ci)
        def _(out_ref=out_ref):
            pltpu.sync_copy(sdeg.at[pl.ds(s * ROWS_T, ROWS_T)],
                            out_ref.at[pl.ds(s * ROWS_T, ROWS_T)])


def _edge_pump(g_ref, src_hbm, dst_hbm, sup0, n_sup,
               sidx_v, didx_v, rows_v, acc, isem, gsem, ssem):
    pltpu.sync_copy(src_hbm.at[sup0], sidx_v.at[0])
    pltpu.sync_copy(dst_hbm.at[sup0], didx_v.at[0])

    def body(jj, carry):
        bank = lax.rem(jj, 2)
        nxt = sup0 + jnp.minimum(jj + 1, n_sup - 1)
        i0 = pltpu.async_copy(src_hbm.at[nxt], sidx_v.at[1 - bank], isem)
        i1 = pltpu.async_copy(dst_hbm.at[nxt], didx_v.at[1 - bank], isem)
        gd = [pltpu.async_copy(g_ref.at[sidx_v.at[bank, t]], rows_v.at[t],
                               gsem) for t in range(_K)]
        for t in range(_K):
            gd[t].wait()
            pltpu.async_copy(rows_v.at[t], acc.at[didx_v.at[bank, t]],
                             ssem, add=True).wait()
        i0.wait()
        i1.wait()
        return carry

    lax.fori_loop(0, n_sup, body, 0)


@functools.partial(
    pl.kernel,
    out_type=(_f32((NP, 128)), _f32((NP, 128))),
    mesh=_MESH,
    scratch_types=[
        pltpu.VMEM((2, _K, EC), jnp.int32),
        pltpu.VMEM((2, _K, EC), jnp.int32),
        pltpu.VMEM((_K, EC, 128), jnp.float32),
        pltpu.VMEM_SHARED((NP, 128), jnp.float32),
        pltpu.SemaphoreType.DMA,
        pltpu.SemaphoreType.DMA,
        pltpu.SemaphoreType.DMA,
    ],
)
def _sc_scatter_split(g_lo, g_hi, src_hbm, dst_hbm, s_lo, s_hi,
                      sidx_v, didx_v, rows_v, acc, isem, gsem, ssem):
    c = lax.axis_index("c")
    s = lax.axis_index("s")
    for ci, (g_ref, s_ref) in enumerate(((g_lo, s_lo), (g_hi, s_hi))):
        @pl.when(c == ci)
        def _(g_ref=g_ref, s_ref=s_ref):
            pltpu.sync_copy(g_ref.at[pl.ds(s * ROWS_T, ROWS_T)],
                            acc.at[pl.ds(s * ROWS_T, ROWS_T)])
            plsc.subcore_barrier()
            _edge_pump(g_ref, src_hbm, dst_hbm, s * SUP_SC, SUP_SC,
                       sidx_v, didx_v, rows_v, acc, isem, gsem, ssem)
            plsc.subcore_barrier()
            pltpu.sync_copy(acc.at[pl.ds(s * ROWS_T, ROWS_T)],
                            s_ref.at[pl.ds(s * ROWS_T, ROWS_T)])


@functools.partial(
    pl.kernel,
    out_type=(_f32((NP, 128)), _f32((NP, 128))),
    mesh=_MESH,
    scratch_types=[
        pltpu.VMEM((2, _K, EC), jnp.int32),
        pltpu.VMEM((2, _K, EC), jnp.int32),
        pltpu.VMEM((_K, EC, 128), jnp.float32),
        pltpu.VMEM_SHARED((NP, 128), jnp.float32),
        pltpu.SemaphoreType.DMA,
        pltpu.SemaphoreType.DMA,
        pltpu.SemaphoreType.DMA,
    ],
)
def _sc_scatter_edges(g_hbm, zeros_hbm, src_hbm, dst_hbm, s_a, s_b,
                      sidx_v, didx_v, rows_v, acc, isem, gsem, ssem):
    c = lax.axis_index("c")
    s = lax.axis_index("s")
    wid = c * NS + s
    for ci, (init_ref, s_ref) in enumerate(((g_hbm, s_a), (zeros_hbm, s_b))):
        @pl.when(c == ci)
        def _(init_ref=init_ref, s_ref=s_ref):
            pltpu.sync_copy(init_ref.at[pl.ds(s * ROWS_T, ROWS_T)],
                            acc.at[pl.ds(s * ROWS_T, ROWS_T)])
            plsc.subcore_barrier()
            _edge_pump(g_hbm, src_hbm, dst_hbm, wid * SUP_ALL, SUP_ALL,
                       sidx_v, didx_v, rows_v, acc, isem, gsem, ssem)
            plsc.subcore_barrier()
            pltpu.sync_copy(acc.at[pl.ds(s * ROWS_T, ROWS_T)],
                            s_ref.at[pl.ds(s * ROWS_T, ROWS_T)])


_IPT = IDX // 32


@functools.partial(
    pl.kernel,
    out_type=_f32((IDX, 64)),
    mesh=_MESH,
    scratch_types=[
        pltpu.VMEM((_IPT,), jnp.int32),
        pltpu.VMEM((_IPT, 128), jnp.float32),
        pltpu.SemaphoreType.DMA,
    ],
)
def _sc_gather_out(h3_hbm, idx_hbm, out_hbm, iidx_v, orows_v, sem):
    c = lax.axis_index("c")
    s = lax.axis_index("s")
    base = (c * NS + s) * _IPT
    pltpu.sync_copy(idx_hbm.at[pl.ds(base, _IPT)], iidx_v)
    pltpu.async_copy(h3_hbm.at[iidx_v], orows_v, sem).wait()

    def body(r, carry):
        pltpu.sync_copy(orows_v.at[r, pl.ds(0, 64)], out_hbm.at[base + r])
        return carry

    lax.fori_loop(0, _IPT, body, 0)


_BM = 1024
_PREC = jax.lax.Precision.HIGHEST


def _tc1_body(x_ref, w_ref, da_ref, db_ref, glo_ref, ghi_ref, dv_ref):
    dv = jax.lax.rsqrt(da_ref[...] + db_ref[...] + 1.0)[:, None]
    m = jnp.dot(x_ref[...], w_ref[...], preferred_element_type=jnp.float32,
                precision=_PREC)
    g = m * dv
    glo_ref[...] = g[:, :128]
    ghi_ref[...] = g[:, 128:]
    dv_ref[...] = dv


def _tc1(x_p, w1, deg_a, deg_b):
    d_in = x_p.shape[1]
    return pl.pallas_call(
        _tc1_body,
        grid=(NP // _BM,),
        in_specs=[
            pl.BlockSpec((_BM, d_in), lambda m: (m, 0)),
            pl.BlockSpec((d_in, 256), lambda m: (0, 0)),
            pl.BlockSpec((_BM,), lambda m: (m,)),
            pl.BlockSpec((_BM,), lambda m: (m,)),
        ],
        out_specs=(
            pl.BlockSpec((_BM, 128), lambda m: (m, 0)),
            pl.BlockSpec((_BM, 128), lambda m: (m, 0)),
            pl.BlockSpec((_BM, 1), lambda m: (m, 0)),
        ),
        out_shape=(_f32((NP, 128)), _f32((NP, 128)), _f32((NP, 1))),
    )(x_p, w1, deg_a, deg_b)


def _tc_mid_body(combine, pad_out, sa_ref, sb_ref, dv_ref, b_ref, w_ref,
                 g_ref):
    dv = dv_ref[...]
    if combine == "concat":
        sfull = jnp.concatenate([sa_ref[...], sb_ref[...]], axis=1)
    else:
        sfull = sa_ref[...] + sb_ref[...]
    h = jnp.maximum(dv * sfull + b_ref[...][None, :], 0.0)
    m = jnp.dot(h, w_ref[...], preferred_element_type=jnp.float32,
                precision=_PREC)
    g = m * dv
    if pad_out:
        g = jnp.concatenate([g, jnp.zeros_like(g)], axis=1)
    g_ref[...] = g


def _tc_mid(s_a, s_b, dinv, b, w, combine):
    dh_in = s_a.shape[1]
    d_in = b.shape[0]
    d_out = w.shape[1]
    pad_out = d_out < 128
    d_store = 128
    return pl.pallas_call(
        functools.partial(_tc_mid_body, combine, pad_out),
        grid=(NP // _BM,),
        in_specs=[
            pl.BlockSpec((_BM, dh_in), lambda m: (m, 0)),
            pl.BlockSpec((_BM, dh_in), lambda m: (m, 0)),
            pl.BlockSpec((_BM, 1), lambda m: (m, 0)),
            pl.BlockSpec((d_in,), lambda m: (0,)),
            pl.BlockSpec((d_in, d_out), lambda m: (0, 0)),
        ],
        out_specs=pl.BlockSpec((_BM, d_store), lambda m: (m, 0)),
        out_shape=_f32((NP, d_store)),
    )(s_a, s_b, dinv, b, w)


def _tc_final_body(sa_ref, sb_ref, dv_ref, b_ref, h_ref):
    sfull = sa_ref[...] + sb_ref[...]
    h_ref[...] = jnp.maximum(dv_ref[...] * sfull + b_ref[...][None, :], 0.0)


def _tc_final(s_a, s_b, dinv, b_pad):
    return pl.pallas_call(
        _tc_final_body,
        grid=(NP // _BM,),
        in_specs=[
            pl.BlockSpec((_BM, 128), lambda m: (m, 0)),
            pl.BlockSpec((_BM, 128), lambda m: (m, 0)),
            pl.BlockSpec((_BM, 1), lambda m: (m, 0)),
            pl.BlockSpec((128,), lambda m: (0,)),
        ],
        out_specs=pl.BlockSpec((_BM, 128), lambda m: (m, 0)),
        out_shape=_f32((NP, 128)),
    )(s_a, s_b, dinv, b_pad)


def kernel(x, edge_index, index, W1, b1, W2, b2, W3, b3):
    pad = jnp.full((EP - E,), PAD_NODE, dtype=jnp.int32)
    src2 = jnp.concatenate([edge_index[0], pad]).reshape(NSUP, _K, EC)
    dst2 = jnp.concatenate([edge_index[1], pad]).reshape(NSUP, _K, EC)
    x_p = jnp.pad(x, ((0, NP - N), (0, 0)))
    zeros_np = jnp.zeros((NP,), jnp.float32)
    zeros_128 = jnp.zeros((NP, 128), jnp.float32)
    b3_pad = jnp.pad(b3, (0, 64))

    deg_a, deg_b = _sc_degree(dst2, zeros_np)
    g1_lo, g1_hi, dinv = _tc1(x_p, W1, deg_a, deg_b)
    s1_lo, s1_hi = _sc_scatter_split(g1_lo, g1_hi, src2, dst2)
    g2 = _tc_mid(s1_lo, s1_hi, dinv, b1, W2, combine="concat")
    s2_a, s2_b = _sc_scatter_edges(g2, zeros_128, src2, dst2)
    g3 = _tc_mid(s2_a, s2_b, dinv, b2, W3, combine="add")
    s3_a, s3_b = _sc_scatter_edges(g3, zeros_128, src2, dst2)
    h3 = _tc_final(s3_a, s3_b, dinv, b3_pad)
    return _sc_gather_out(h3, index)


# speedup vs baseline: 17.9811x; 2.1349x over previous
import functools

import jax
import jax.numpy as jnp
from jax import lax
from jax.experimental import pallas as pl
from jax.experimental.pallas import tpu as pltpu
from jax.experimental.pallas import tpu_sc as plsc

N = 10000
E = 160000
NP = 10240
EP = 163840
EC = 64
EROWS = EP // EC
_K = 5
NSUP = EROWS // _K
IDX = 2048
PAD_NODE = N

NC = 2
NS = 16
ROWS_T = NP // NS
SUP_SC = NSUP // NS
SUP_ALL = NSUP // (NC * NS)

_MESH = plsc.VectorSubcoreMesh(core_axis_name="c", subcore_axis_name="s")


def _f32(shape):
    return jax.ShapeDtypeStruct(shape, jnp.float32)


@functools.partial(
    pl.kernel,
    out_type=(_f32((NP,)), _f32((NP,))),
    mesh=_MESH,
    scratch_types=[
        pltpu.VMEM((SUP_ALL, _K, EC), jnp.int32),
        pltpu.VMEM((EC,), jnp.float32),
        pltpu.VMEM_SHARED((NP,), jnp.float32),
    ],
)
def _sc_degree(dst_hbm, zeros_hbm, deg_a, deg_b, didx_v, ones_v, sdeg):
    c = lax.axis_index("c")
    s = lax.axis_index("s")
    wid = c * NS + s
    for i in range(EC // 16):
        ones_v[pl.ds(i * 16, 16)] = jnp.ones((16,), jnp.float32)

    @pl.when(s == 0)
    def _():
        pltpu.sync_copy(zeros_hbm, sdeg)

    pltpu.sync_copy(dst_hbm.at[pl.ds(wid * SUP_ALL, SUP_ALL)], didx_v)
    plsc.subcore_barrier()

    def body(j, carry):
        for t in range(_K):
            pltpu.sync_copy(ones_v, sdeg.at[didx_v.at[j, t]], add=True)
        return carry

    lax.fori_loop(0, SUP_ALL, body, 0)
    plsc.subcore_barrier()
    for ci, out_ref in enumerate((deg_a, deg_b)):
        @pl.when(c == ci)
        def _(out_ref=out_ref):
            pltpu.sync_copy(sdeg.at[pl.ds(s * ROWS_T, ROWS_T)],
                            out_ref.at[pl.ds(s * ROWS_T, ROWS_T)])


def _edge_pump(g_ref, src_hbm, dst_hbm, sup0, n_sup,
               sidx_v, didx_v, rows_v, acc, isem, gsem, ssem):
    pltpu.sync_copy(src_hbm.at[sup0], sidx_v.at[0])
    pltpu.sync_copy(dst_hbm.at[sup0], didx_v.at[0])

    def body(jj, carry):
        bank = lax.rem(jj, 2)
        nxt = sup0 + jnp.minimum(jj + 1, n_sup - 1)
        i0 = pltpu.async_copy(src_hbm.at[nxt], sidx_v.at[1 - bank], isem)
        i1 = pltpu.async_copy(dst_hbm.at[nxt], didx_v.at[1 - bank], isem)
        gd = [pltpu.async_copy(g_ref.at[sidx_v.at[bank, t]], rows_v.at[t],
                               gsem) for t in range(_K)]
        for t in range(_K):
            gd[t].wait()
            pltpu.async_copy(rows_v.at[t], acc.at[didx_v.at[bank, t]],
                             ssem, add=True).wait()
        i0.wait()
        i1.wait()
        return carry

    lax.fori_loop(0, n_sup, body, 0)


@functools.partial(
    pl.kernel,
    out_type=(_f32((NP, 128)), _f32((NP, 128))),
    mesh=_MESH,
    scratch_types=[
        pltpu.VMEM((2, _K, EC), jnp.int32),
        pltpu.VMEM((2, _K, EC), jnp.int32),
        pltpu.VMEM((_K, EC, 128), jnp.float32),
        pltpu.VMEM_SHARED((NP, 128), jnp.float32),
        pltpu.SemaphoreType.DMA,
        pltpu.SemaphoreType.DMA,
        pltpu.SemaphoreType.DMA,
    ],
)
def _sc_scatter_split(g_lo, g_hi, src_hbm, dst_hbm, s_lo, s_hi,
                      sidx_v, didx_v, rows_v, acc, isem, gsem, ssem):
    c = lax.axis_index("c")
    s = lax.axis_index("s")
    for ci, (g_ref, s_ref) in enumerate(((g_lo, s_lo), (g_hi, s_hi))):
        @pl.when(c == ci)
        def _(g_ref=g_ref, s_ref=s_ref):
            pltpu.sync_copy(g_ref.at[pl.ds(s * ROWS_T, ROWS_T)],
                            acc.at[pl.ds(s * ROWS_T, ROWS_T)])
            plsc.subcore_barrier()
            _edge_pump(g_ref, src_hbm, dst_hbm, s * SUP_SC, SUP_SC,
                       sidx_v, didx_v, rows_v, acc, isem, gsem, ssem)
            plsc.subcore_barrier()
            pltpu.sync_copy(acc.at[pl.ds(s * ROWS_T, ROWS_T)],
                            s_ref.at[pl.ds(s * ROWS_T, ROWS_T)])


@functools.partial(
    pl.kernel,
    out_type=(_f32((NP, 128)), _f32((NP, 128))),
    mesh=_MESH,
    scratch_types=[
        pltpu.VMEM((2, _K, EC), jnp.int32),
        pltpu.VMEM((2, _K, EC), jnp.int32),
        pltpu.VMEM((_K, EC, 128), jnp.float32),
        pltpu.VMEM_SHARED((NP, 128), jnp.float32),
        pltpu.SemaphoreType.DMA,
        pltpu.SemaphoreType.DMA,
        pltpu.SemaphoreType.DMA,
    ],
)
def _sc_scatter_edges(g_hbm, zeros_hbm, src_hbm, dst_hbm, s_a, s_b,
                      sidx_v, didx_v, rows_v, acc, isem, gsem, ssem):
    c = lax.axis_index("c")
    s = lax.axis_index("s")
    wid = c * NS + s
    for ci, (init_ref, s_ref) in enumerate(((g_hbm, s_a), (zeros_hbm, s_b))):
        @pl.when(c == ci)
        def _(init_ref=init_ref, s_ref=s_ref):
            pltpu.sync_copy(init_ref.at[pl.ds(s * ROWS_T, ROWS_T)],
                            acc.at[pl.ds(s * ROWS_T, ROWS_T)])
            plsc.subcore_barrier()
            _edge_pump(g_hbm, src_hbm, dst_hbm, wid * SUP_ALL, SUP_ALL,
                       sidx_v, didx_v, rows_v, acc, isem, gsem, ssem)
            plsc.subcore_barrier()
            pltpu.sync_copy(acc.at[pl.ds(s * ROWS_T, ROWS_T)],
                            s_ref.at[pl.ds(s * ROWS_T, ROWS_T)])


_IPT = IDX // 32


@functools.partial(
    pl.kernel,
    out_type=_f32((IDX, 64)),
    mesh=_MESH,
    scratch_types=[
        pltpu.VMEM((_IPT,), jnp.int32),
        pltpu.VMEM((_IPT, 128), jnp.float32),
        pltpu.SemaphoreType.DMA,
    ],
)
def _sc_gather_out(h3_hbm, idx_hbm, out_hbm, iidx_v, orows_v, sem):
    c = lax.axis_index("c")
    s = lax.axis_index("s")
    base = (c * NS + s) * _IPT
    pltpu.sync_copy(idx_hbm.at[pl.ds(base, _IPT)], iidx_v)
    pltpu.async_copy(h3_hbm.at[iidx_v], orows_v, sem).wait()

    def body(r, carry):
        pltpu.sync_copy(orows_v.at[r, pl.ds(0, 64)], out_hbm.at[base + r])
        return carry

    lax.fori_loop(0, _IPT, body, 0)


_BM = 1024
_PREC = jax.lax.Precision.HIGHEST


def _tc1_body(x_ref, w_ref, da_ref, db_ref, glo_ref, ghi_ref, dv_ref):
    dv = jax.lax.rsqrt(da_ref[...] + db_ref[...] + 1.0)[:, None]
    m = jnp.dot(x_ref[...], w_ref[...], preferred_element_type=jnp.float32,
                precision=_PREC)
    g = m * dv
    glo_ref[...] = g[:, :128]
    ghi_ref[...] = g[:, 128:]
    dv_ref[...] = dv


def _tc1(x_p, w1, deg_a, deg_b):
    d_in = x_p.shape[1]
    return pl.pallas_call(
        _tc1_body,
        grid=(NP // _BM,),
        in_specs=[
            pl.BlockSpec((_BM, d_in), lambda m: (m, 0)),
            pl.BlockSpec((d_in, 256), lambda m: (0, 0)),
            pl.BlockSpec((_BM,), lambda m: (m,)),
            pl.BlockSpec((_BM,), lambda m: (m,)),
        ],
        out_specs=(
            pl.BlockSpec((_BM, 128), lambda m: (m, 0)),
            pl.BlockSpec((_BM, 128), lambda m: (m, 0)),
            pl.BlockSpec((_BM, 1), lambda m: (m, 0)),
        ),
        out_shape=(_f32((NP, 128)), _f32((NP, 128)), _f32((NP, 1))),
    )(x_p, w1, deg_a, deg_b)


def _tc_mid_body(combine, pad_out, sa_ref, sb_ref, dv_ref, b_ref, w_ref,
                 g_ref):
    dv = dv_ref[...]
    if combine == "concat":
        sfull = jnp.concatenate([sa_ref[...], sb_ref[...]], axis=1)
    else:
        sfull = sa_ref[...] + sb_ref[...]
    h = jnp.maximum(dv * sfull + b_ref[...][None, :], 0.0)
    m = jnp.dot(h, w_ref[...], preferred_element_type=jnp.float32,
                precision=_PREC)
    g = m * dv
    if pad_out:
        g = jnp.concatenate([g, jnp.zeros_like(g)], axis=1)
    g_ref[...] = g


def _tc_mid(s_a, s_b, dinv, b, w, combine):
    dh_in = s_a.shape[1]
    d_in = b.shape[0]
    d_out = w.shape[1]
    pad_out = d_out < 128
    d_store = 128
    return pl.pallas_call(
        functools.partial(_tc_mid_body, combine, pad_out),
        grid=(NP // _BM,),
        in_specs=[
            pl.BlockSpec((_BM, dh_in), lambda m: (m, 0)),
            pl.BlockSpec((_BM, dh_in), lambda m: (m, 0)),
            pl.BlockSpec((_BM, 1), lambda m: (m, 0)),
            pl.BlockSpec((d_in,), lambda m: (0,)),
            pl.BlockSpec((d_in, d_out), lambda m: (0, 0)),
        ],
        out_specs=pl.BlockSpec((_BM, d_store), lambda m: (m, 0)),
        out_shape=_f32((NP, d_store)),
    )(s_a, s_b, dinv, b, w)


def _tc_final_body(sa_ref, sb_ref, dv_ref, b_ref, h_ref):
    sfull = sa_ref[...] + sb_ref[...]
    h_ref[...] = jnp.maximum(dv_ref[...] * sfull + b_ref[...][None, :], 0.0)


def _tc_final(s_a, s_b, dinv, b_pad):
    return pl.pallas_call(
        _tc_final_body,
        grid=(NP // _BM,),
        in_specs=[
            pl.BlockSpec((_BM, 128), lambda m: (m, 0)),
            pl.BlockSpec((_BM, 128), lambda m: (m, 0)),
            pl.BlockSpec((_BM, 1), lambda m: (m, 0)),
            pl.BlockSpec((128,), lambda m: (0,)),
        ],
        out_specs=pl.BlockSpec((_BM, 128), lambda m: (m, 0)),
        out_shape=_f32((NP, 128)),
    )(s_a, s_b, dinv, b_pad)


def kernel(x, edge_index, index, W1, b1, W2, b2, W3, b3):
    pad = (jnp.arange(EP - E, dtype=jnp.int32) % (NP - N)) + PAD_NODE
    src2 = jnp.concatenate([edge_index[0], pad]).reshape(NSUP, _K, EC)
    dst2 = jnp.concatenate([edge_index[1], pad]).reshape(NSUP, _K, EC)
    x_p = jnp.pad(x, ((0, NP - N), (0, 0)))
    zeros_np = jnp.zeros((NP,), jnp.float32)
    zeros_128 = jnp.zeros((NP, 128), jnp.float32)
    b3_pad = jnp.pad(b3, (0, 64))

    deg_a, deg_b = _sc_degree(dst2, zeros_np)
    g1_lo, g1_hi, dinv = _tc1(x_p, W1, deg_a, deg_b)
    s1_lo, s1_hi = _sc_scatter_split(g1_lo, g1_hi, src2, dst2)
    g2 = _tc_mid(s1_lo, s1_hi, dinv, b1, W2, combine="concat")
    s2_a, s2_b = _sc_scatter_edges(g2, zeros_128, src2, dst2)
    g3 = _tc_mid(s2_a, s2_b, dinv, b2, W3, combine="add")
    s3_a, s3_b = _sc_scatter_edges(g3, zeros_128, src2, dst2)
    h3 = _tc_final(s3_a, s3_b, dinv, b3_pad)
    return _sc_gather_out(h3, index)


# speedup vs baseline: 18.0607x; 1.0044x over previous
import functools

import jax
import jax.numpy as jnp
from jax import lax
from jax.experimental import pallas as pl
from jax.experimental.pallas import tpu as pltpu
from jax.experimental.pallas import tpu_sc as plsc

N = 10000
E = 160000
NP = 10240
EP = 163840
EC = 64
EROWS = EP // EC
_K = 5
NSUP = EROWS // _K
IDX = 2048
PAD_NODE = N

NC = 2
NS = 16
ROWS_T = NP // NS
SUP_SC = NSUP // NS
SUP_ALL = NSUP // (NC * NS)

_MESH = plsc.VectorSubcoreMesh(core_axis_name="c", subcore_axis_name="s")


def _f32(shape):
    return jax.ShapeDtypeStruct(shape, jnp.float32)


@functools.partial(
    pl.kernel,
    out_type=(_f32((NP,)), _f32((NP,))),
    mesh=_MESH,
    scratch_types=[
        pltpu.VMEM((SUP_ALL, _K, EC), jnp.int32),
        pltpu.VMEM((EC,), jnp.float32),
        pltpu.VMEM_SHARED((NP,), jnp.float32),
    ],
)
def _sc_degree(dst_hbm, zeros_hbm, deg_a, deg_b, didx_v, ones_v, sdeg):
    c = lax.axis_index("c")
    s = lax.axis_index("s")
    wid = c * NS + s
    for i in range(EC // 16):
        ones_v[pl.ds(i * 16, 16)] = jnp.ones((16,), jnp.float32)

    @pl.when(s == 0)
    def _():
        pltpu.sync_copy(zeros_hbm, sdeg)

    pltpu.sync_copy(dst_hbm.at[pl.ds(wid * SUP_ALL, SUP_ALL)], didx_v)
    plsc.subcore_barrier()

    def body(j, carry):
        for t in range(_K):
            pltpu.sync_copy(ones_v, sdeg.at[didx_v.at[j, t]], add=True)
        return carry

    lax.fori_loop(0, SUP_ALL, body, 0)
    plsc.subcore_barrier()
    for ci, out_ref in enumerate((deg_a, deg_b)):
        @pl.when(c == ci)
        def _(out_ref=out_ref):
            pltpu.sync_copy(sdeg.at[pl.ds(s * ROWS_T, ROWS_T)],
                            out_ref.at[pl.ds(s * ROWS_T, ROWS_T)])


def _edge_pump(g_ref, src_hbm, dst_hbm, sup0, n_sup,
               sidx_v, didx_v, rows_v, acc, isem, gsem, ssem):
    pltpu.sync_copy(src_hbm.at[sup0], sidx_v.at[0])
    pltpu.sync_copy(dst_hbm.at[sup0], didx_v.at[0])

    def body(jj, carry):
        bank = lax.rem(jj, 2)
        nxt = sup0 + jnp.minimum(jj + 1, n_sup - 1)
        i0 = pltpu.async_copy(src_hbm.at[nxt], sidx_v.at[1 - bank], isem)
        i1 = pltpu.async_copy(dst_hbm.at[nxt], didx_v.at[1 - bank], isem)
        gd = [pltpu.async_copy(g_ref.at[sidx_v.at[bank, t]], rows_v.at[t],
                               gsem) for t in range(_K)]
        for t in range(_K):
            gd[t].wait()
            pltpu.async_copy(rows_v.at[t], acc.at[didx_v.at[bank, t]],
                             ssem, add=True).wait()
        i0.wait()
        i1.wait()
        return carry

    lax.fori_loop(0, n_sup, body, 0)


@functools.partial(
    pl.kernel,
    out_type=(_f32((NP, 128)), _f32((NP, 128))),
    mesh=_MESH,
    scratch_types=[
        pltpu.VMEM((2, _K, EC), jnp.int32),
        pltpu.VMEM((2, _K, EC), jnp.int32),
        pltpu.VMEM((_K, EC, 128), jnp.float32),
        pltpu.VMEM_SHARED((NP, 128), jnp.float32),
        pltpu.SemaphoreType.DMA,
        pltpu.SemaphoreType.DMA,
        pltpu.SemaphoreType.DMA,
    ],
)
def _sc_scatter_split(g_lo, g_hi, src_hbm, dst_hbm, s_lo, s_hi,
                      sidx_v, didx_v, rows_v, acc, isem, gsem, ssem):
    c = lax.axis_index("c")
    s = lax.axis_index("s")
    for ci, (g_ref, s_ref) in enumerate(((g_lo, s_lo), (g_hi, s_hi))):
        @pl.when(c == ci)
        def _(g_ref=g_ref, s_ref=s_ref):
            pltpu.sync_copy(g_ref.at[pl.ds(s * ROWS_T, ROWS_T)],
                            acc.at[pl.ds(s * ROWS_T, ROWS_T)])
            plsc.subcore_barrier()
            _edge_pump(g_ref, src_hbm, dst_hbm, s * SUP_SC, SUP_SC,
                       sidx_v, didx_v, rows_v, acc, isem, gsem, ssem)
            plsc.subcore_barrier()
            pltpu.sync_copy(acc.at[pl.ds(s * ROWS_T, ROWS_T)],
                            s_ref.at[pl.ds(s * ROWS_T, ROWS_T)])


@functools.partial(
    pl.kernel,
    out_type=(_f32((NP, 128)), _f32((NP, 128))),
    mesh=_MESH,
    scratch_types=[
        pltpu.VMEM((2, _K, EC), jnp.int32),
        pltpu.VMEM((2, _K, EC), jnp.int32),
        pltpu.VMEM((_K, EC, 128), jnp.float32),
        pltpu.VMEM_SHARED((NP, 128), jnp.float32),
        pltpu.SemaphoreType.DMA,
        pltpu.SemaphoreType.DMA,
        pltpu.SemaphoreType.DMA,
    ],
)
def _sc_scatter_edges(g_hbm, zeros_hbm, src_hbm, dst_hbm, s_a, s_b,
                      sidx_v, didx_v, rows_v, acc, isem, gsem, ssem):
    c = lax.axis_index("c")
    s = lax.axis_index("s")
    wid = c * NS + s
    for ci, (init_ref, s_ref) in enumerate(((g_hbm, s_a), (zeros_hbm, s_b))):
        @pl.when(c == ci)
        def _(init_ref=init_ref, s_ref=s_ref):
            pltpu.sync_copy(init_ref.at[pl.ds(s * ROWS_T, ROWS_T)],
                            acc.at[pl.ds(s * ROWS_T, ROWS_T)])
            plsc.subcore_barrier()
            _edge_pump(g_hbm, src_hbm, dst_hbm, wid * SUP_ALL, SUP_ALL,
                       sidx_v, didx_v, rows_v, acc, isem, gsem, ssem)
            plsc.subcore_barrier()
            pltpu.sync_copy(acc.at[pl.ds(s * ROWS_T, ROWS_T)],
                            s_ref.at[pl.ds(s * ROWS_T, ROWS_T)])


_IPT = IDX // 32


@functools.partial(
    pl.kernel,
    out_type=_f32((IDX, 64)),
    mesh=_MESH,
    compiler_params=pltpu.CompilerParams(needs_layout_passes=False),
    scratch_types=[
        pltpu.VMEM((_IPT,), jnp.int32),
        pltpu.VMEM((_IPT, 128), jnp.float32),
        pltpu.VMEM((_IPT, 128), jnp.float32),
        pltpu.VMEM((NP // 128, 128), jnp.float32),
        pltpu.VMEM((64,), jnp.float32),
        pltpu.VMEM((_IPT, 64), jnp.float32),
        pltpu.SemaphoreType.DMA,
        pltpu.SemaphoreType.DMA,
    ],
)
def _sc_out_fused(s3a_hbm, s3b_hbm, dinv_hbm, b3_hbm, idx_hbm, out_hbm,
                  iidx_v, rowsa_v, rowsb_v, dv_v, b3_v, out_v, sema, semb):
    c = lax.axis_index("c")
    s = lax.axis_index("s")
    base = (c * NS + s) * _IPT
    pltpu.sync_copy(idx_hbm.at[pl.ds(base, _IPT)], iidx_v)
    pltpu.sync_copy(dinv_hbm, dv_v)
    pltpu.sync_copy(b3_hbm, b3_v)
    ga = pltpu.async_copy(s3a_hbm.at[iidx_v], rowsa_v, sema)
    gb = pltpu.async_copy(s3b_hbm.at[iidx_v], rowsb_v, semb)
    ga.wait()
    gb.wait()

    def fbody(f, carry):
        f16 = jnp.full((16,), f, jnp.int32)
        b3f = plsc.load_gather(b3_v, [f16])
        for q in range(_IPT // 16):
            r16 = jnp.arange(16, dtype=jnp.int32) + q * 16
            idx16 = iidx_v[pl.ds(q * 16, 16)]
            dv16 = plsc.load_gather(
                dv_v, [lax.shift_right_logical(idx16, 7),
                       lax.bitwise_and(idx16, jnp.int32(127))])
            a = plsc.load_gather(rowsa_v, [r16, f16])
            b = plsc.load_gather(rowsb_v, [r16, f16])
            res = jnp.maximum(dv16 * (a + b) + b3f, 0.0)
            plsc.store_scatter(out_v, [r16, f16], res)
        return carry

    lax.fori_loop(0, 64, fbody, 0)
    pltpu.sync_copy(out_v, out_hbm.at[pl.ds(base, _IPT)])


_BM = 1024
_PREC = jax.lax.Precision.HIGHEST


def _tc1_body(x_ref, w_ref, da_ref, db_ref, glo_ref, ghi_ref, dv_ref):
    dv = jax.lax.rsqrt(da_ref[...] + db_ref[...] + 1.0)[:, None]
    m = jnp.dot(x_ref[...], w_ref[...], preferred_element_type=jnp.float32,
                precision=_PREC)
    g = m * dv
    glo_ref[...] = g[:, :128]
    ghi_ref[...] = g[:, 128:]
    dv_ref[...] = dv


def _tc1(x_p, w1, deg_a, deg_b):
    d_in = x_p.shape[1]
    return pl.pallas_call(
        _tc1_body,
        grid=(NP // _BM,),
        in_specs=[
            pl.BlockSpec((_BM, d_in), lambda m: (m, 0)),
            pl.BlockSpec((d_in, 256), lambda m: (0, 0)),
            pl.BlockSpec((_BM,), lambda m: (m,)),
            pl.BlockSpec((_BM,), lambda m: (m,)),
        ],
        out_specs=(
            pl.BlockSpec((_BM, 128), lambda m: (m, 0)),
            pl.BlockSpec((_BM, 128), lambda m: (m, 0)),
            pl.BlockSpec((_BM, 1), lambda m: (m, 0)),
        ),
        out_shape=(_f32((NP, 128)), _f32((NP, 128)), _f32((NP, 1))),
    )(x_p, w1, deg_a, deg_b)


def _tc_mid_body(combine, pad_out, sa_ref, sb_ref, dv_ref, b_ref, w_ref,
                 g_ref):
    dv = dv_ref[...]
    if combine == "concat":
        sfull = jnp.concatenate([sa_ref[...], sb_ref[...]], axis=1)
    else:
        sfull = sa_ref[...] + sb_ref[...]
    h = jnp.maximum(dv * sfull + b_ref[...][None, :], 0.0)
    m = jnp.dot(h, w_ref[...], preferred_element_type=jnp.float32,
                precision=_PREC)
    g = m * dv
    if pad_out:
        g = jnp.concatenate([g, jnp.zeros_like(g)], axis=1)
    g_ref[...] = g


def _tc_mid(s_a, s_b, dinv, b, w, combine):
    dh_in = s_a.shape[1]
    d_in = b.shape[0]
    d_out = w.shape[1]
    pad_out = d_out < 128
    d_store = 128
    return pl.pallas_call(
        functools.partial(_tc_mid_body, combine, pad_out),
        grid=(NP // _BM,),
        in_specs=[
            pl.BlockSpec((_BM, dh_in), lambda m: (m, 0)),
            pl.BlockSpec((_BM, dh_in), lambda m: (m, 0)),
            pl.BlockSpec((_BM, 1), lambda m: (m, 0)),
            pl.BlockSpec((d_in,), lambda m: (0,)),
            pl.BlockSpec((d_in, d_out), lambda m: (0, 0)),
        ],
        out_specs=pl.BlockSpec((_BM, d_store), lambda m: (m, 0)),
        out_shape=_f32((NP, d_store)),
    )(s_a, s_b, dinv, b, w)


def kernel(x, edge_index, index, W1, b1, W2, b2, W3, b3):
    pad = (jnp.arange(EP - E, dtype=jnp.int32) % (NP - N)) + PAD_NODE
    src2 = jnp.concatenate([edge_index[0], pad]).reshape(NSUP, _K, EC)
    dst2 = jnp.concatenate([edge_index[1], pad]).reshape(NSUP, _K, EC)
    x_p = jnp.pad(x, ((0, NP - N), (0, 0)))
    zeros_np = jnp.zeros((NP,), jnp.float32)
    zeros_128 = jnp.zeros((NP, 128), jnp.float32)

    deg_a, deg_b = _sc_degree(dst2, zeros_np)
    g1_lo, g1_hi, dinv = _tc1(x_p, W1, deg_a, deg_b)
    s1_lo, s1_hi = _sc_scatter_split(g1_lo, g1_hi, src2, dst2)
    g2 = _tc_mid(s1_lo, s1_hi, dinv, b1, W2, combine="concat")
    s2_a, s2_b = _sc_scatter_edges(g2, zeros_128, src2, dst2)
    g3 = _tc_mid(s2_a, s2_b, dinv, b2, W3, combine="add")
    s3_a, s3_b = _sc_scatter_edges(g3, zeros_128, src2, dst2)
    return _sc_out_fused(s3_a, s3_b, dinv.reshape(NP // 128, 128), b3, index)


# speedup vs baseline: 18.4499x; 1.0215x over previous
import functools

import jax
import jax.numpy as jnp
from jax import lax
from jax.experimental import pallas as pl
from jax.experimental.pallas import tpu as pltpu
from jax.experimental.pallas import tpu_sc as plsc

N = 10000
E = 160000
NP = 10240
EP = 163840
EC = 64
EROWS = EP // EC
_K = 5
NSUP = EROWS // _K
IDX = 2048
PAD_NODE = N

NC = 2
NS = 16
ROWS_T = NP // NS
SUP_SC = NSUP // NS
SUP_ALL = NSUP // (NC * NS)

_MESH = plsc.VectorSubcoreMesh(core_axis_name="c", subcore_axis_name="s")


def _f32(shape):
    return jax.ShapeDtypeStruct(shape, jnp.float32)


@functools.partial(
    pl.kernel,
    out_type=(_f32((NP,)), _f32((NP,))),
    mesh=_MESH,
    scratch_types=[
        pltpu.VMEM((SUP_ALL, _K, EC), jnp.int32),
        pltpu.VMEM((EC,), jnp.float32),
        pltpu.VMEM_SHARED((NP,), jnp.float32),
    ],
)
def _sc_degree(dst_hbm, zeros_hbm, deg_a, deg_b, didx_v, ones_v, sdeg):
    c = lax.axis_index("c")
    s = lax.axis_index("s")
    wid = c * NS + s
    for i in range(EC // 16):
        ones_v[pl.ds(i * 16, 16)] = jnp.ones((16,), jnp.float32)

    @pl.when(s == 0)
    def _():
        pltpu.sync_copy(zeros_hbm, sdeg)

    pltpu.sync_copy(dst_hbm.at[pl.ds(wid * SUP_ALL, SUP_ALL)], didx_v)
    plsc.subcore_barrier()

    def body(j, carry):
        for t in range(_K):
            pltpu.sync_copy(ones_v, sdeg.at[didx_v.at[j, t]], add=True)
        return carry

    lax.fori_loop(0, SUP_ALL, body, 0)
    plsc.subcore_barrier()
    for ci, out_ref in enumerate((deg_a, deg_b)):
        @pl.when(c == ci)
        def _(out_ref=out_ref):
            pltpu.sync_copy(sdeg.at[pl.ds(s * ROWS_T, ROWS_T)],
                            out_ref.at[pl.ds(s * ROWS_T, ROWS_T)])


def _edge_pump(g_ref, src_hbm, dst_hbm, sup0, n_sup,
               sidx_v, didx_v, rows_v, acc, isem, gsem, ssem):
    pltpu.sync_copy(src_hbm.at[sup0], sidx_v.at[0])
    pltpu.sync_copy(dst_hbm.at[sup0], didx_v.at[0])

    def body(jj, carry):
        bank = lax.rem(jj, 2)
        nxt = sup0 + jnp.minimum(jj + 1, n_sup - 1)
        i0 = pltpu.async_copy(src_hbm.at[nxt], sidx_v.at[1 - bank], isem)
        i1 = pltpu.async_copy(dst_hbm.at[nxt], didx_v.at[1 - bank], isem)
        gd = [pltpu.async_copy(g_ref.at[sidx_v.at[bank, t]], rows_v.at[t],
                               gsem) for t in range(_K)]
        for t in range(_K):
            gd[t].wait()
            pltpu.async_copy(rows_v.at[t], acc.at[didx_v.at[bank, t]],
                             ssem, add=True).wait()
        i0.wait()
        i1.wait()
        return carry

    lax.fori_loop(0, n_sup, body, 0)


@functools.partial(
    pl.kernel,
    out_type=(_f32((NP, 128)), _f32((NP, 128))),
    mesh=_MESH,
    scratch_types=[
        pltpu.VMEM((2, _K, EC), jnp.int32),
        pltpu.VMEM((2, _K, EC), jnp.int32),
        pltpu.VMEM((_K, EC, 128), jnp.float32),
        pltpu.VMEM_SHARED((NP, 128), jnp.float32),
        pltpu.SemaphoreType.DMA,
        pltpu.SemaphoreType.DMA,
        pltpu.SemaphoreType.DMA,
    ],
)
def _sc_scatter_split(g_lo, g_hi, src_hbm, dst_hbm, s_lo, s_hi,
                      sidx_v, didx_v, rows_v, acc, isem, gsem, ssem):
    c = lax.axis_index("c")
    s = lax.axis_index("s")
    for ci, (g_ref, s_ref) in enumerate(((g_lo, s_lo), (g_hi, s_hi))):
        @pl.when(c == ci)
        def _(g_ref=g_ref, s_ref=s_ref):
            pltpu.sync_copy(g_ref.at[pl.ds(s * ROWS_T, ROWS_T)],
                            acc.at[pl.ds(s * ROWS_T, ROWS_T)])
            plsc.subcore_barrier()
            _edge_pump(g_ref, src_hbm, dst_hbm, s * SUP_SC, SUP_SC,
                       sidx_v, didx_v, rows_v, acc, isem, gsem, ssem)
            plsc.subcore_barrier()
            pltpu.sync_copy(acc.at[pl.ds(s * ROWS_T, ROWS_T)],
                            s_ref.at[pl.ds(s * ROWS_T, ROWS_T)])


@functools.partial(
    pl.kernel,
    out_type=(_f32((NP, 128)), _f32((NP, 128))),
    mesh=_MESH,
    scratch_types=[
        pltpu.VMEM((2, _K, EC), jnp.int32),
        pltpu.VMEM((2, _K, EC), jnp.int32),
        pltpu.VMEM((_K, EC, 128), jnp.float32),
        pltpu.VMEM_SHARED((NP, 128), jnp.float32),
        pltpu.SemaphoreType.DMA,
        pltpu.SemaphoreType.DMA,
        pltpu.SemaphoreType.DMA,
    ],
)
def _sc_scatter_edges(g_hbm, zeros_hbm, src_hbm, dst_hbm, s_a, s_b,
                      sidx_v, didx_v, rows_v, acc, isem, gsem, ssem):
    c = lax.axis_index("c")
    s = lax.axis_index("s")
    wid = c * NS + s
    for ci, (init_ref, s_ref) in enumerate(((g_hbm, s_a), (zeros_hbm, s_b))):
        @pl.when(c == ci)
        def _(init_ref=init_ref, s_ref=s_ref):
            pltpu.sync_copy(init_ref.at[pl.ds(s * ROWS_T, ROWS_T)],
                            acc.at[pl.ds(s * ROWS_T, ROWS_T)])
            plsc.subcore_barrier()
            _edge_pump(g_hbm, src_hbm, dst_hbm, wid * SUP_ALL, SUP_ALL,
                       sidx_v, didx_v, rows_v, acc, isem, gsem, ssem)
            plsc.subcore_barrier()
            pltpu.sync_copy(acc.at[pl.ds(s * ROWS_T, ROWS_T)],
                            s_ref.at[pl.ds(s * ROWS_T, ROWS_T)])


_IPT = IDX // 32


@functools.partial(
    pl.kernel,
    out_type=_f32((IDX, 64)),
    mesh=_MESH,
    compiler_params=pltpu.CompilerParams(needs_layout_passes=False),
    scratch_types=[
        pltpu.VMEM((_IPT,), jnp.int32),
        pltpu.VMEM((_IPT, 128), jnp.float32),
        pltpu.VMEM((_IPT, 128), jnp.float32),
        pltpu.VMEM((NP // 128, 128), jnp.float32),
        pltpu.VMEM((64,), jnp.float32),
        pltpu.VMEM((_IPT, 64), jnp.float32),
        pltpu.SemaphoreType.DMA,
        pltpu.SemaphoreType.DMA,
    ],
)
def _sc_out_fused(s3a_hbm, s3b_hbm, dinv_hbm, b3_hbm, idx_hbm, out_hbm,
                  iidx_v, rowsa_v, rowsb_v, dv_v, b3_v, out_v, sema, semb):
    c = lax.axis_index("c")
    s = lax.axis_index("s")
    base = (c * NS + s) * _IPT
    pltpu.sync_copy(idx_hbm.at[pl.ds(base, _IPT)], iidx_v)
    pltpu.sync_copy(dinv_hbm, dv_v)
    pltpu.sync_copy(b3_hbm, b3_v)
    ga = pltpu.async_copy(s3a_hbm.at[iidx_v], rowsa_v, sema)
    gb = pltpu.async_copy(s3b_hbm.at[iidx_v], rowsb_v, semb)
    ga.wait()
    gb.wait()

    def fbody(f, carry):
        f16 = jnp.full((16,), f, jnp.int32)
        b3f = plsc.load_gather(b3_v, [f16])
        for q in range(_IPT // 16):
            r16 = jnp.arange(16, dtype=jnp.int32) + q * 16
            idx16 = iidx_v[pl.ds(q * 16, 16)]
            dv16 = plsc.load_gather(
                dv_v, [lax.shift_right_logical(idx16, 7),
                       lax.bitwise_and(idx16, jnp.int32(127))])
            a = plsc.load_gather(rowsa_v, [r16, f16])
            b = plsc.load_gather(rowsb_v, [r16, f16])
            res = jnp.maximum(dv16 * (a + b) + b3f, 0.0)
            plsc.store_scatter(out_v, [r16, f16], res)
        return carry

    lax.fori_loop(0, 64, fbody, 0)
    pltpu.sync_copy(out_v, out_hbm.at[pl.ds(base, _IPT)])


_BM = 1024
_PREC = jax.lax.Precision.DEFAULT


def _tc1_body(x_ref, w_ref, da_ref, db_ref, glo_ref, ghi_ref, dv_ref):
    dv = jax.lax.rsqrt(da_ref[...] + db_ref[...] + 1.0)[:, None]
    m = jnp.dot(x_ref[...], w_ref[...], preferred_element_type=jnp.float32,
                precision=_PREC)
    g = m * dv
    glo_ref[...] = g[:, :128]
    ghi_ref[...] = g[:, 128:]
    dv_ref[...] = dv


def _tc1(x_p, w1, deg_a, deg_b):
    d_in = x_p.shape[1]
    return pl.pallas_call(
        _tc1_body,
        grid=(NP // _BM,),
        in_specs=[
            pl.BlockSpec((_BM, d_in), lambda m: (m, 0)),
            pl.BlockSpec((d_in, 256), lambda m: (0, 0)),
            pl.BlockSpec((_BM,), lambda m: (m,)),
            pl.BlockSpec((_BM,), lambda m: (m,)),
        ],
        out_specs=(
            pl.BlockSpec((_BM, 128), lambda m: (m, 0)),
            pl.BlockSpec((_BM, 128), lambda m: (m, 0)),
            pl.BlockSpec((_BM, 1), lambda m: (m, 0)),
        ),
        out_shape=(_f32((NP, 128)), _f32((NP, 128)), _f32((NP, 1))),
    )(x_p, w1, deg_a, deg_b)


def _tc_mid_body(combine, pad_out, sa_ref, sb_ref, dv_ref, b_ref, w_ref,
                 g_ref):
    dv = dv_ref[...]
    if combine == "concat":
        sfull = jnp.concatenate([sa_ref[...], sb_ref[...]], axis=1)
    else:
        sfull = sa_ref[...] + sb_ref[...]
    h = jnp.maximum(dv * sfull + b_ref[...][None, :], 0.0)
    m = jnp.dot(h, w_ref[...], preferred_element_type=jnp.float32,
                precision=_PREC)
    g = m * dv
    if pad_out:
        g = jnp.concatenate([g, jnp.zeros_like(g)], axis=1)
    g_ref[...] = g


def _tc_mid(s_a, s_b, dinv, b, w, combine):
    dh_in = s_a.shape[1]
    d_in = b.shape[0]
    d_out = w.shape[1]
    pad_out = d_out < 128
    d_store = 128
    return pl.pallas_call(
        functools.partial(_tc_mid_body, combine, pad_out),
        grid=(NP // _BM,),
        in_specs=[
            pl.BlockSpec((_BM, dh_in), lambda m: (m, 0)),
            pl.BlockSpec((_BM, dh_in), lambda m: (m, 0)),
            pl.BlockSpec((_BM, 1), lambda m: (m, 0)),
            pl.BlockSpec((d_in,), lambda m: (0,)),
            pl.BlockSpec((d_in, d_out), lambda m: (0, 0)),
        ],
        out_specs=pl.BlockSpec((_BM, d_store), lambda m: (m, 0)),
        out_shape=_f32((NP, d_store)),
    )(s_a, s_b, dinv, b, w)


def kernel(x, edge_index, index, W1, b1, W2, b2, W3, b3):
    pad = (jnp.arange(EP - E, dtype=jnp.int32) % (NP - N)) + PAD_NODE
    src2 = jnp.concatenate([edge_index[0], pad]).reshape(NSUP, _K, EC)
    dst2 = jnp.concatenate([edge_index[1], pad]).reshape(NSUP, _K, EC)
    x_p = jnp.pad(x, ((0, NP - N), (0, 0)))
    zeros_np = jnp.zeros((NP,), jnp.float32)
    zeros_128 = jnp.zeros((NP, 128), jnp.float32)

    deg_a, deg_b = _sc_degree(dst2, zeros_np)
    g1_lo, g1_hi, dinv = _tc1(x_p, W1, deg_a, deg_b)
    s1_lo, s1_hi = _sc_scatter_split(g1_lo, g1_hi, src2, dst2)
    g2 = _tc_mid(s1_lo, s1_hi, dinv, b1, W2, combine="concat")
    s2_a, s2_b = _sc_scatter_edges(g2, zeros_128, src2, dst2)
    g3 = _tc_mid(s2_a, s2_b, dinv, b2, W3, combine="add")
    s3_a, s3_b = _sc_scatter_edges(g3, zeros_128, src2, dst2)
    return _sc_out_fused(s3_a, s3_b, dinv.reshape(NP // 128, 128), b3, index)
